# baseline (speedup 1.0000x reference)
import functools

import jax
import jax.numpy as jnp
from jax.experimental import pallas as pl
from jax.experimental.pallas import tpu as pltpu

_VMEM_LIMIT = 48 * 1024 * 1024
_BF = jnp.bfloat16


def _conv1_body(x_ref, we_ref, wo_ref, b_ref, o_ref):
    lhs_t = x_ref[0]
    dn = (((0,), (0,)), ((), ()))
    oe = jax.lax.dot_general(lhs_t, we_ref[...], dn,
                             preferred_element_type=jnp.float32)
    oo = jax.lax.dot_general(lhs_t, wo_ref[...], dn,
                             preferred_element_type=jnp.float32)
    z = jnp.maximum(jnp.maximum(oe, oo) + b_ref[...], 0.0)
    z = z.reshape(64, 2, 8, 128)
    z = jnp.maximum(z[:, 0], z[:, 1])
    o_ref[0] = z.astype(o_ref.dtype)


def _conv1(x_nchw, w3, b):
    B = x_nchw.shape[0]
    xp = jnp.pad(x_nchw, ((0, 0), (0, 0), (1, 1), (1, 1))).astype(_BF)
    slabs = []
    for dh in range(3):
        for t in range(18):
            s = xp[:, :, dh:dh + 128, t:t + 113:16]
            slabs.append(s.reshape(B, 3, 1024))
    cols = jnp.concatenate(slabs, axis=1)

    wz = jnp.zeros((2, 3, 18, 3, 8, 16), _BF)
    w3b = w3.astype(_BF)
    for m in range(8):
        for dw in range(3):
            wz = wz.at[0, :, 2 * m + dw, :, m, :].set(w3b[:, dw])
            wz = wz.at[1, :, 2 * m + 1 + dw, :, m, :].set(w3b[:, dw])
    we = wz[0].reshape(162, 128)
    wo = wz[1].reshape(162, 128)
    b128 = jnp.tile(b, (1, 8))

    return pl.pallas_call(
        _conv1_body,
        out_shape=jax.ShapeDtypeStruct((B, 64, 8, 128), _BF),
        grid=(B,),
        in_specs=[
            pl.BlockSpec((1, 162, 1024), lambda i: (i, 0, 0)),
            pl.BlockSpec((162, 128), lambda i: (0, 0)),
            pl.BlockSpec((162, 128), lambda i: (0, 0)),
            pl.BlockSpec((1, 128), lambda i: (0, 0)),
        ],
        out_specs=pl.BlockSpec((1, 64, 8, 128), lambda i: (i, 0, 0, 0)),
        compiler_params=pltpu.CompilerParams(
            dimension_semantics=("parallel",),
            vmem_limit_bytes=_VMEM_LIMIT,
        ),
    )(cols, we, wo, b128)


def _conv_eo_body(ce_ref, co_ref, w_ref, b_ref, o_ref, *, H, wh, k3, co):
    rows = H * wh
    acc_e = None
    acc_o = None
    for dh in range(3):
        le = ce_ref[0, dh:dh + H].reshape(rows, k3)
        lo = co_ref[0, dh:dh + H].reshape(rows, k3)
        pe = jnp.dot(le, w_ref[dh], preferred_element_type=jnp.float32)
        po = jnp.dot(lo, w_ref[dh], preferred_element_type=jnp.float32)
        acc_e = pe if acc_e is None else acc_e + pe
        acc_o = po if acc_o is None else acc_o + po
    z = jnp.maximum(jnp.maximum(acc_e, acc_o) + b_ref[...], 0.0)
    z = z.reshape(H // 2, 2, wh, co)
    z = jnp.maximum(z[:, 0], z[:, 1])
    o_ref[0] = z.astype(o_ref.dtype)


def _conv_eo(x, w3, b):
    B, H, W, C = x.shape
    Co = w3.shape[-1]
    k3 = 3 * C
    wh = W // 2
    xp = jnp.pad(x, ((0, 0), (1, 1), (1, 1), (0, 0)))
    ce = jnp.concatenate([xp[:, :, d:d + W:2, :] for d in range(3)], axis=-1)
    cod = jnp.concatenate([xp[:, :, d + 1:d + 1 + W:2, :] for d in range(3)],
                          axis=-1)
    wk = w3.astype(_BF).reshape(3, k3, Co)
    body = functools.partial(_conv_eo_body, H=H, wh=wh, k3=k3, co=Co)
    return pl.pallas_call(
        body,
        out_shape=jax.ShapeDtypeStruct((B, H // 2, wh, Co), _BF),
        grid=(B,),
        in_specs=[
            pl.BlockSpec((1, H + 2, wh, k3), lambda i: (i, 0, 0, 0)),
            pl.BlockSpec((1, H + 2, wh, k3), lambda i: (i, 0, 0, 0)),
            pl.BlockSpec((3, k3, Co), lambda i: (0, 0, 0)),
            pl.BlockSpec((1, Co), lambda i: (0, 0)),
        ],
        out_specs=pl.BlockSpec((1, H // 2, wh, Co), lambda i: (i, 0, 0, 0)),
        compiler_params=pltpu.CompilerParams(
            dimension_semantics=("parallel",),
            vmem_limit_bytes=_VMEM_LIMIT,
        ),
    )(ce, cod, wk, b)


def _mlp_body(x_ref, w1_ref, b1_ref, w2_ref, b2_ref, o_ref):
    h = jnp.dot(x_ref[...], w1_ref[...], preferred_element_type=jnp.float32)
    h = jnp.maximum(h + b1_ref[...], 0.0)
    o_ref[...] = jnp.dot(h, w2_ref[...], preferred_element_type=jnp.float32) + b2_ref[...]


def _mlp(x, w1, b1, w2, b2):
    B, K = x.shape
    N1 = w1.shape[1]
    N2 = w2.shape[1]
    return pl.pallas_call(
        _mlp_body,
        out_shape=jax.ShapeDtypeStruct((B, N2), jnp.float32),
        grid=(1,),
        in_specs=[
            pl.BlockSpec((B, K), lambda i: (0, 0)),
            pl.BlockSpec((K, N1), lambda i: (0, 0)),
            pl.BlockSpec((1, N1), lambda i: (0, 0)),
            pl.BlockSpec((N1, N2), lambda i: (0, 0)),
            pl.BlockSpec((1, N2), lambda i: (0, 0)),
        ],
        out_specs=pl.BlockSpec((B, N2), lambda i: (0, 0)),
        compiler_params=pltpu.CompilerParams(
            dimension_semantics=("arbitrary",),
            vmem_limit_bytes=_VMEM_LIMIT,
        ),
    )(x, w1, b1, w2, b2)


def kernel(x_nchw, wc1, bc1, wc2, bc2, wc3, bc3, wl1, bl1, wl2, bl2):
    B = x_nchw.shape[0]
    x = _conv1(x_nchw, wc1, bc1)
    x = x.reshape(B, 64, 64, 16)
    x = _conv_eo(x, wc2, bc2)
    x = _conv_eo(x, wc3, bc3)
    x = x.reshape(B, -1)
    return _mlp(x, wl1.astype(_BF), bl1, wl2, bl2)


# speedup vs baseline: 1.7322x; 1.7322x over previous
import functools

import jax
import jax.numpy as jnp
from jax.experimental import pallas as pl
from jax.experimental.pallas import tpu as pltpu

_VMEM_LIMIT = 48 * 1024 * 1024
_BF = jnp.bfloat16


def _conv1_body(x_ref, we_ref, wo_ref, b_ref, o_ref):
    lhs_t = x_ref[0]
    dn = (((0,), (0,)), ((), ()))
    oe = jax.lax.dot_general(lhs_t, we_ref[...], dn,
                             preferred_element_type=jnp.float32)
    oo = jax.lax.dot_general(lhs_t, wo_ref[...], dn,
                             preferred_element_type=jnp.float32)
    z = jnp.maximum(jnp.maximum(oe, oo) + b_ref[...], 0.0)
    z = z.reshape(64, 2, 8, 128)
    z = jnp.maximum(z[:, 0], z[:, 1])
    o_ref[0] = z.astype(o_ref.dtype)


def _conv1(x_nchw, w3, b):
    B = x_nchw.shape[0]
    xp = jnp.pad(x_nchw, ((0, 0), (0, 0), (1, 1), (1, 15))).astype(_BF)
    xg = xp.reshape(B, 3, 130, 9, 16)
    w18 = jnp.concatenate([xg[:, :, :, 0:8, :], xg[:, :, :, 1:9, 0:2]],
                          axis=-1)
    rows = jnp.stack([w18[:, :, dh:dh + 128] for dh in range(3)], axis=1)
    cols = rows.transpose(0, 1, 5, 2, 3, 4).reshape(B, 162, 1024)

    w3b = w3.astype(_BF)
    we = jnp.stack([jnp.pad(w3b, ((0, 0), (2 * m, 15 - 2 * m), (0, 0), (0, 0)))
                    for m in range(8)], axis=3)
    wo = jnp.stack([jnp.pad(w3b, ((0, 0), (2 * m + 1, 14 - 2 * m), (0, 0), (0, 0)))
                    for m in range(8)], axis=3)
    we = we.reshape(162, 128)
    wo = wo.reshape(162, 128)
    b128 = jnp.tile(b, (1, 8))

    return pl.pallas_call(
        _conv1_body,
        out_shape=jax.ShapeDtypeStruct((B, 64, 8, 128), _BF),
        grid=(B,),
        in_specs=[
            pl.BlockSpec((1, 162, 1024), lambda i: (i, 0, 0)),
            pl.BlockSpec((162, 128), lambda i: (0, 0)),
            pl.BlockSpec((162, 128), lambda i: (0, 0)),
            pl.BlockSpec((1, 128), lambda i: (0, 0)),
        ],
        out_specs=pl.BlockSpec((1, 64, 8, 128), lambda i: (i, 0, 0, 0)),
        compiler_params=pltpu.CompilerParams(
            dimension_semantics=("parallel",),
            vmem_limit_bytes=_VMEM_LIMIT,
        ),
    )(cols, we, wo, b128)


def _conv_eo_body(ce_ref, co_ref, w_ref, b_ref, o_ref, *, H, wh, k3, co):
    rows = H * wh
    acc_e = None
    acc_o = None
    for dh in range(3):
        le = ce_ref[0, dh:dh + H].reshape(rows, k3)
        lo = co_ref[0, dh:dh + H].reshape(rows, k3)
        pe = jnp.dot(le, w_ref[dh], preferred_element_type=jnp.float32)
        po = jnp.dot(lo, w_ref[dh], preferred_element_type=jnp.float32)
        acc_e = pe if acc_e is None else acc_e + pe
        acc_o = po if acc_o is None else acc_o + po
    z = jnp.maximum(jnp.maximum(acc_e, acc_o) + b_ref[...], 0.0)
    z = z.reshape(H // 2, 2, wh, co)
    z = jnp.maximum(z[:, 0], z[:, 1])
    o_ref[0] = z.astype(o_ref.dtype)


def _conv_eo(x, w3, b):
    B, H, W, C = x.shape
    Co = w3.shape[-1]
    k3 = 3 * C
    wh = W // 2
    xp = jnp.pad(x, ((0, 0), (1, 1), (1, 1), (0, 0)))
    ce = jnp.concatenate([xp[:, :, d:d + W:2, :] for d in range(3)], axis=-1)
    cod = jnp.concatenate([xp[:, :, d + 1:d + 1 + W:2, :] for d in range(3)],
                          axis=-1)
    wk = w3.astype(_BF).reshape(3, k3, Co)
    body = functools.partial(_conv_eo_body, H=H, wh=wh, k3=k3, co=Co)
    return pl.pallas_call(
        body,
        out_shape=jax.ShapeDtypeStruct((B, H // 2, wh, Co), _BF),
        grid=(B,),
        in_specs=[
            pl.BlockSpec((1, H + 2, wh, k3), lambda i: (i, 0, 0, 0)),
            pl.BlockSpec((1, H + 2, wh, k3), lambda i: (i, 0, 0, 0)),
            pl.BlockSpec((3, k3, Co), lambda i: (0, 0, 0)),
            pl.BlockSpec((1, Co), lambda i: (0, 0)),
        ],
        out_specs=pl.BlockSpec((1, H // 2, wh, Co), lambda i: (i, 0, 0, 0)),
        compiler_params=pltpu.CompilerParams(
            dimension_semantics=("parallel",),
            vmem_limit_bytes=_VMEM_LIMIT,
        ),
    )(ce, cod, wk, b)


def _mlp_body(x_ref, w1_ref, b1_ref, w2_ref, b2_ref, o_ref):
    h = jnp.dot(x_ref[...], w1_ref[...], preferred_element_type=jnp.float32)
    h = jnp.maximum(h + b1_ref[...], 0.0)
    o_ref[...] = jnp.dot(h, w2_ref[...], preferred_element_type=jnp.float32) + b2_ref[...]


def _mlp(x, w1, b1, w2, b2):
    B, K = x.shape
    N1 = w1.shape[1]
    N2 = w2.shape[1]
    return pl.pallas_call(
        _mlp_body,
        out_shape=jax.ShapeDtypeStruct((B, N2), jnp.float32),
        grid=(1,),
        in_specs=[
            pl.BlockSpec((B, K), lambda i: (0, 0)),
            pl.BlockSpec((K, N1), lambda i: (0, 0)),
            pl.BlockSpec((1, N1), lambda i: (0, 0)),
            pl.BlockSpec((N1, N2), lambda i: (0, 0)),
            pl.BlockSpec((1, N2), lambda i: (0, 0)),
        ],
        out_specs=pl.BlockSpec((B, N2), lambda i: (0, 0)),
        compiler_params=pltpu.CompilerParams(
            dimension_semantics=("arbitrary",),
            vmem_limit_bytes=_VMEM_LIMIT,
        ),
    )(x, w1, b1, w2, b2)


def kernel(x_nchw, wc1, bc1, wc2, bc2, wc3, bc3, wl1, bl1, wl2, bl2):
    B = x_nchw.shape[0]
    x = _conv1(x_nchw, wc1, bc1)
    x = x.reshape(B, 64, 64, 16)
    x = _conv_eo(x, wc2, bc2)
    x = _conv_eo(x, wc3, bc3)
    x = x.reshape(B, -1)
    return _mlp(x, wl1.astype(_BF), bl1, wl2, bl2)


# speedup vs baseline: 3.4260x; 1.9778x over previous
import functools

import jax
import jax.numpy as jnp
from jax.experimental import pallas as pl
from jax.experimental.pallas import tpu as pltpu

_VMEM_LIMIT = 48 * 1024 * 1024
_BF = jnp.bfloat16


def _conv1_body(x_ref, we_ref, wo_ref, b_ref, o_ref):
    lhs_t = x_ref[0]
    dn = (((0,), (0,)), ((), ()))
    oe = jax.lax.dot_general(lhs_t, we_ref[...], dn,
                             preferred_element_type=jnp.float32)
    oo = jax.lax.dot_general(lhs_t, wo_ref[...], dn,
                             preferred_element_type=jnp.float32)
    z = jnp.maximum(jnp.maximum(oe, oo) + b_ref[...], 0.0)
    z = z.reshape(64, 2, 8, 128)
    z = jnp.maximum(z[:, 0], z[:, 1])
    o_ref[0] = z.astype(o_ref.dtype)


def _conv1(x_nchw, w3, b):
    B = x_nchw.shape[0]
    xp = jnp.pad(x_nchw, ((0, 0), (0, 0), (1, 1), (1, 15))).astype(_BF)
    xg = xp.reshape(B, 3, 130, 9, 16)
    w18 = jnp.concatenate([xg[:, :, :, 0:8, :], xg[:, :, :, 1:9, 0:2]],
                          axis=-1)
    rows = jnp.stack([w18[:, :, dh:dh + 128] for dh in range(3)], axis=1)
    cols = rows.transpose(0, 1, 5, 2, 3, 4).reshape(B, 162, 1024)

    w3b = w3.astype(_BF)
    we = jnp.stack([jnp.pad(w3b, ((0, 0), (2 * m, 15 - 2 * m), (0, 0), (0, 0)))
                    for m in range(8)], axis=3)
    wo = jnp.stack([jnp.pad(w3b, ((0, 0), (2 * m + 1, 14 - 2 * m), (0, 0), (0, 0)))
                    for m in range(8)], axis=3)
    we = we.reshape(162, 128)
    wo = wo.reshape(162, 128)
    b128 = jnp.tile(b, (1, 8))

    return pl.pallas_call(
        _conv1_body,
        out_shape=jax.ShapeDtypeStruct((B, 64, 8, 128), _BF),
        grid=(B,),
        in_specs=[
            pl.BlockSpec((1, 162, 1024), lambda i: (i, 0, 0)),
            pl.BlockSpec((162, 128), lambda i: (0, 0)),
            pl.BlockSpec((162, 128), lambda i: (0, 0)),
            pl.BlockSpec((1, 128), lambda i: (0, 0)),
        ],
        out_specs=pl.BlockSpec((1, 64, 8, 128), lambda i: (i, 0, 0, 0)),
        compiler_params=pltpu.CompilerParams(
            dimension_semantics=("parallel",),
            vmem_limit_bytes=_VMEM_LIMIT,
        ),
    )(cols, we, wo, b128)


def _packed_body(x_ref, we_ref, wo_ref, b_ref, o_ref, *, H, G, N):
    lhs_t = x_ref[0]
    dn = (((0,), (0,)), ((), ()))
    oe = jax.lax.dot_general(lhs_t, we_ref[...], dn,
                             preferred_element_type=jnp.float32)
    oo = jax.lax.dot_general(lhs_t, wo_ref[...], dn,
                             preferred_element_type=jnp.float32)
    z = jnp.maximum(jnp.maximum(oe, oo) + b_ref[...], 0.0)
    z = z.reshape(H // 2, 2, G, N)
    z = jnp.maximum(z[:, 0], z[:, 1])
    o_ref[0] = z.astype(o_ref.dtype)


def _packed_conv(x, w3, b, *, p):
    B, H, W, C = x.shape
    Co = w3.shape[-1]
    G = W // p
    K = 3 * (p + 2) * C
    N = (p // 2) * Co
    xp = jnp.pad(x, ((0, 0), (1, 1), (1, (G + 1) * p - W - 1), (0, 0)))
    xg = xp.reshape(B, H + 2, G + 1, p, C)
    win = jnp.concatenate([xg[:, :, 0:G], xg[:, :, 1:G + 1, 0:2]],
                          axis=3)
    rows = jnp.stack([win[:, dh:dh + H] for dh in range(3)], axis=1)
    cols = rows.transpose(0, 1, 4, 5, 2, 3).reshape(B, K, H * G)

    w3b = w3.astype(_BF)
    we = jnp.stack([jnp.pad(w3b, ((0, 0), (2 * m, p - 1 - 2 * m), (0, 0), (0, 0)))
                    for m in range(p // 2)], axis=3)
    wo = jnp.stack([jnp.pad(w3b, ((0, 0), (2 * m + 1, p - 2 - 2 * m), (0, 0), (0, 0)))
                    for m in range(p // 2)], axis=3)
    we = we.reshape(K, N)
    wo = wo.reshape(K, N)
    bN = jnp.tile(b, (1, p // 2))

    body = functools.partial(_packed_body, H=H, G=G, N=N)
    return pl.pallas_call(
        body,
        out_shape=jax.ShapeDtypeStruct((B, H // 2, G, N), _BF),
        grid=(B,),
        in_specs=[
            pl.BlockSpec((1, K, H * G), lambda i: (i, 0, 0)),
            pl.BlockSpec((K, N), lambda i: (0, 0)),
            pl.BlockSpec((K, N), lambda i: (0, 0)),
            pl.BlockSpec((1, N), lambda i: (0, 0)),
        ],
        out_specs=pl.BlockSpec((1, H // 2, G, N), lambda i: (i, 0, 0, 0)),
        compiler_params=pltpu.CompilerParams(
            dimension_semantics=("parallel",),
            vmem_limit_bytes=_VMEM_LIMIT,
        ),
    )(cols, we, wo, bN)


def _mlp_body(x_ref, w1_ref, b1_ref, w2_ref, b2_ref, o_ref):
    h = jnp.dot(x_ref[...], w1_ref[...], preferred_element_type=jnp.float32)
    h = jnp.maximum(h + b1_ref[...], 0.0)
    o_ref[...] = jnp.dot(h, w2_ref[...], preferred_element_type=jnp.float32) + b2_ref[...]


def _mlp(x, w1, b1, w2, b2):
    B, K = x.shape
    N1 = w1.shape[1]
    N2 = w2.shape[1]
    return pl.pallas_call(
        _mlp_body,
        out_shape=jax.ShapeDtypeStruct((B, N2), jnp.float32),
        grid=(1,),
        in_specs=[
            pl.BlockSpec((B, K), lambda i: (0, 0)),
            pl.BlockSpec((K, N1), lambda i: (0, 0)),
            pl.BlockSpec((1, N1), lambda i: (0, 0)),
            pl.BlockSpec((N1, N2), lambda i: (0, 0)),
            pl.BlockSpec((1, N2), lambda i: (0, 0)),
        ],
        out_specs=pl.BlockSpec((B, N2), lambda i: (0, 0)),
        compiler_params=pltpu.CompilerParams(
            dimension_semantics=("arbitrary",),
            vmem_limit_bytes=_VMEM_LIMIT,
        ),
    )(x, w1, b1, w2, b2)


def kernel(x_nchw, wc1, bc1, wc2, bc2, wc3, bc3, wl1, bl1, wl2, bl2):
    B = x_nchw.shape[0]
    x = _conv1(x_nchw, wc1, bc1)
    x = x.reshape(B, 64, 64, 16)
    x = _packed_conv(x, wc2, bc2, p=16)
    x = x.reshape(B, 32, 32, 32)
    x = _packed_conv(x, wc3, bc3, p=8)
    x = x.reshape(B, -1)
    return _mlp(x, wl1.astype(_BF), bl1, wl2, bl2)


# speedup vs baseline: 4.3070x; 1.2571x over previous
import functools

import jax
import jax.numpy as jnp
from jax.experimental import pallas as pl
from jax.experimental.pallas import tpu as pltpu

_VMEM_LIMIT = 48 * 1024 * 1024
_BF = jnp.bfloat16


_BT = 8


def _conv1_body(x_ref, we_ref, wo_ref, b_ref, o_ref):
    dn = (((0,), (0,)), ((), ()))
    for bi in range(_BT):
        lhs_t = x_ref[bi]
        oe = jax.lax.dot_general(lhs_t, we_ref[...], dn,
                                 preferred_element_type=jnp.float32)
        oo = jax.lax.dot_general(lhs_t, wo_ref[...], dn,
                                 preferred_element_type=jnp.float32)
        z = jnp.maximum(jnp.maximum(oe, oo) + b_ref[...], 0.0)
        z = z.reshape(64, 2, 8, 128)
        z = jnp.maximum(z[:, 0], z[:, 1])
        o_ref[bi] = z.astype(o_ref.dtype)


def _conv1(x_nchw, w3, b):
    B = x_nchw.shape[0]
    xp = jnp.pad(x_nchw, ((0, 0), (0, 0), (1, 1), (1, 15))).astype(_BF)
    xg = xp.reshape(B, 3, 130, 9, 16)
    w18 = jnp.concatenate([xg[:, :, :, 0:8, :], xg[:, :, :, 1:9, 0:2]],
                          axis=-1)
    rows = jnp.stack([w18[:, :, dh:dh + 128] for dh in range(3)], axis=1)
    cols = rows.transpose(0, 1, 5, 2, 3, 4).reshape(B, 162, 1024)

    w3b = w3.astype(_BF)
    we = jnp.stack([jnp.pad(w3b, ((0, 0), (2 * m, 15 - 2 * m), (0, 0), (0, 0)))
                    for m in range(8)], axis=3)
    wo = jnp.stack([jnp.pad(w3b, ((0, 0), (2 * m + 1, 14 - 2 * m), (0, 0), (0, 0)))
                    for m in range(8)], axis=3)
    we = we.reshape(162, 128)
    wo = wo.reshape(162, 128)
    b128 = jnp.tile(b, (1, 8))

    return pl.pallas_call(
        _conv1_body,
        out_shape=jax.ShapeDtypeStruct((B, 64, 8, 128), _BF),
        grid=(B // _BT,),
        in_specs=[
            pl.BlockSpec((_BT, 162, 1024), lambda i: (i, 0, 0)),
            pl.BlockSpec((162, 128), lambda i: (0, 0)),
            pl.BlockSpec((162, 128), lambda i: (0, 0)),
            pl.BlockSpec((1, 128), lambda i: (0, 0)),
        ],
        out_specs=pl.BlockSpec((_BT, 64, 8, 128), lambda i: (i, 0, 0, 0)),
        compiler_params=pltpu.CompilerParams(
            dimension_semantics=("parallel",),
            vmem_limit_bytes=_VMEM_LIMIT,
        ),
    )(cols, we, wo, b128)


def _packed_body(x_ref, we_ref, wo_ref, b_ref, o_ref, *, H, G, N):
    dn = (((0,), (0,)), ((), ()))
    for bi in range(_BT):
        lhs_t = x_ref[bi]
        oe = jax.lax.dot_general(lhs_t, we_ref[...], dn,
                                 preferred_element_type=jnp.float32)
        oo = jax.lax.dot_general(lhs_t, wo_ref[...], dn,
                                 preferred_element_type=jnp.float32)
        z = jnp.maximum(jnp.maximum(oe, oo) + b_ref[...], 0.0)
        z = z.reshape(H // 2, 2, G, N)
        z = jnp.maximum(z[:, 0], z[:, 1])
        o_ref[bi] = z.astype(o_ref.dtype)


def _packed_conv(x, w3, b, *, p):
    B, H, W, C = x.shape
    Co = w3.shape[-1]
    G = W // p
    K = 3 * (p + 2) * C
    N = (p // 2) * Co
    xp = jnp.pad(x, ((0, 0), (1, 1), (1, (G + 1) * p - W - 1), (0, 0)))
    xg = xp.reshape(B, H + 2, G + 1, p, C)
    win = jnp.concatenate([xg[:, :, 0:G], xg[:, :, 1:G + 1, 0:2]],
                          axis=3)
    rows = jnp.stack([win[:, dh:dh + H] for dh in range(3)], axis=1)
    cols = rows.transpose(0, 1, 4, 5, 2, 3).reshape(B, K, H * G)

    w3b = w3.astype(_BF)
    we = jnp.stack([jnp.pad(w3b, ((0, 0), (2 * m, p - 1 - 2 * m), (0, 0), (0, 0)))
                    for m in range(p // 2)], axis=3)
    wo = jnp.stack([jnp.pad(w3b, ((0, 0), (2 * m + 1, p - 2 - 2 * m), (0, 0), (0, 0)))
                    for m in range(p // 2)], axis=3)
    we = we.reshape(K, N)
    wo = wo.reshape(K, N)
    bN = jnp.tile(b, (1, p // 2))

    body = functools.partial(_packed_body, H=H, G=G, N=N)
    return pl.pallas_call(
        body,
        out_shape=jax.ShapeDtypeStruct((B, H // 2, G, N), _BF),
        grid=(B // _BT,),
        in_specs=[
            pl.BlockSpec((_BT, K, H * G), lambda i: (i, 0, 0)),
            pl.BlockSpec((K, N), lambda i: (0, 0)),
            pl.BlockSpec((K, N), lambda i: (0, 0)),
            pl.BlockSpec((1, N), lambda i: (0, 0)),
        ],
        out_specs=pl.BlockSpec((_BT, H // 2, G, N), lambda i: (i, 0, 0, 0)),
        compiler_params=pltpu.CompilerParams(
            dimension_semantics=("parallel",),
            vmem_limit_bytes=_VMEM_LIMIT,
        ),
    )(cols, we, wo, bN)


def _mlp_body(x_ref, w1_ref, b1_ref, w2_ref, b2_ref, o_ref):
    h = jnp.dot(x_ref[...], w1_ref[...], preferred_element_type=jnp.float32)
    h = jnp.maximum(h + b1_ref[...], 0.0)
    o_ref[...] = jnp.dot(h, w2_ref[...], preferred_element_type=jnp.float32) + b2_ref[...]


def _mlp(x, w1, b1, w2, b2):
    B, K = x.shape
    N1 = w1.shape[1]
    N2 = w2.shape[1]
    return pl.pallas_call(
        _mlp_body,
        out_shape=jax.ShapeDtypeStruct((B, N2), jnp.float32),
        grid=(1,),
        in_specs=[
            pl.BlockSpec((B, K), lambda i: (0, 0)),
            pl.BlockSpec((K, N1), lambda i: (0, 0)),
            pl.BlockSpec((1, N1), lambda i: (0, 0)),
            pl.BlockSpec((N1, N2), lambda i: (0, 0)),
            pl.BlockSpec((1, N2), lambda i: (0, 0)),
        ],
        out_specs=pl.BlockSpec((B, N2), lambda i: (0, 0)),
        compiler_params=pltpu.CompilerParams(
            dimension_semantics=("arbitrary",),
            vmem_limit_bytes=_VMEM_LIMIT,
        ),
    )(x, w1, b1, w2, b2)


def kernel(x_nchw, wc1, bc1, wc2, bc2, wc3, bc3, wl1, bl1, wl2, bl2):
    B = x_nchw.shape[0]
    x = _conv1(x_nchw, wc1, bc1)
    x = x.reshape(B, 64, 64, 16)
    x = _packed_conv(x, wc2, bc2, p=16)
    x = x.reshape(B, 32, 32, 32)
    x = _packed_conv(x, wc3, bc3, p=8)
    x = x.reshape(B, -1)
    return _mlp(x, wl1.astype(_BF), bl1, wl2, bl2)


# speedup vs baseline: 4.3352x; 1.0066x over previous
import functools

import jax
import jax.numpy as jnp
from jax.experimental import pallas as pl
from jax.experimental.pallas import tpu as pltpu

_VMEM_LIMIT = 48 * 1024 * 1024
_BF = jnp.bfloat16


_BT = 16


def _conv1_body(x_ref, we_ref, wo_ref, b_ref, o_ref):
    dn = (((0,), (0,)), ((), ()))
    for bi in range(_BT):
        lhs_t = x_ref[bi]
        oe = jax.lax.dot_general(lhs_t, we_ref[...], dn,
                                 preferred_element_type=jnp.float32)
        oo = jax.lax.dot_general(lhs_t, wo_ref[...], dn,
                                 preferred_element_type=jnp.float32)
        z = jnp.maximum(jnp.maximum(oe, oo) + b_ref[...], 0.0)
        z = z.reshape(64, 2, 8, 128)
        z = jnp.maximum(z[:, 0], z[:, 1])
        o_ref[bi] = z.astype(o_ref.dtype)


def _conv1(x_nchw, w3, b):
    B = x_nchw.shape[0]
    xp = jnp.pad(x_nchw, ((0, 0), (0, 0), (1, 1), (1, 15))).astype(_BF)
    xg = xp.reshape(B, 3, 130, 9, 16)
    w18 = jnp.concatenate([xg[:, :, :, 0:8, :], xg[:, :, :, 1:9, 0:2]],
                          axis=-1)
    rows = jnp.stack([w18[:, :, dh:dh + 128] for dh in range(3)], axis=1)
    cols = rows.transpose(0, 1, 5, 2, 3, 4).reshape(B, 162, 1024)

    w3b = w3.astype(_BF)
    we = jnp.stack([jnp.pad(w3b, ((0, 0), (2 * m, 15 - 2 * m), (0, 0), (0, 0)))
                    for m in range(8)], axis=3)
    wo = jnp.stack([jnp.pad(w3b, ((0, 0), (2 * m + 1, 14 - 2 * m), (0, 0), (0, 0)))
                    for m in range(8)], axis=3)
    we = we.reshape(162, 128)
    wo = wo.reshape(162, 128)
    b128 = jnp.tile(b, (1, 8))

    return pl.pallas_call(
        _conv1_body,
        out_shape=jax.ShapeDtypeStruct((B, 64, 8, 128), _BF),
        grid=(B // _BT,),
        in_specs=[
            pl.BlockSpec((_BT, 162, 1024), lambda i: (i, 0, 0)),
            pl.BlockSpec((162, 128), lambda i: (0, 0)),
            pl.BlockSpec((162, 128), lambda i: (0, 0)),
            pl.BlockSpec((1, 128), lambda i: (0, 0)),
        ],
        out_specs=pl.BlockSpec((_BT, 64, 8, 128), lambda i: (i, 0, 0, 0)),
        compiler_params=pltpu.CompilerParams(
            dimension_semantics=("parallel",),
            vmem_limit_bytes=_VMEM_LIMIT,
        ),
    )(cols, we, wo, b128)


def _packed_body(x_ref, we_ref, wo_ref, b_ref, o_ref, *, H, G, N):
    dn = (((0,), (0,)), ((), ()))
    for bi in range(_BT):
        lhs_t = x_ref[bi]
        oe = jax.lax.dot_general(lhs_t, we_ref[...], dn,
                                 preferred_element_type=jnp.float32)
        oo = jax.lax.dot_general(lhs_t, wo_ref[...], dn,
                                 preferred_element_type=jnp.float32)
        z = jnp.maximum(jnp.maximum(oe, oo) + b_ref[...], 0.0)
        z = z.reshape(H // 2, 2, G, N)
        z = jnp.maximum(z[:, 0], z[:, 1])
        o_ref[bi] = z.astype(o_ref.dtype)


def _packed_conv(x, w3, b, *, p):
    B, H, W, C = x.shape
    Co = w3.shape[-1]
    G = W // p
    K = 3 * (p + 2) * C
    N = (p // 2) * Co
    xp = jnp.pad(x, ((0, 0), (1, 1), (1, (G + 1) * p - W - 1), (0, 0)))
    xg = xp.reshape(B, H + 2, G + 1, p, C)
    win = jnp.concatenate([xg[:, :, 0:G], xg[:, :, 1:G + 1, 0:2]],
                          axis=3)
    rows = jnp.stack([win[:, dh:dh + H] for dh in range(3)], axis=1)
    cols = rows.transpose(0, 1, 4, 5, 2, 3).reshape(B, K, H * G)

    w3b = w3.astype(_BF)
    we = jnp.stack([jnp.pad(w3b, ((0, 0), (2 * m, p - 1 - 2 * m), (0, 0), (0, 0)))
                    for m in range(p // 2)], axis=3)
    wo = jnp.stack([jnp.pad(w3b, ((0, 0), (2 * m + 1, p - 2 - 2 * m), (0, 0), (0, 0)))
                    for m in range(p // 2)], axis=3)
    we = we.reshape(K, N)
    wo = wo.reshape(K, N)
    bN = jnp.tile(b, (1, p // 2))

    body = functools.partial(_packed_body, H=H, G=G, N=N)
    return pl.pallas_call(
        body,
        out_shape=jax.ShapeDtypeStruct((B, H // 2, G, N), _BF),
        grid=(B // _BT,),
        in_specs=[
            pl.BlockSpec((_BT, K, H * G), lambda i: (i, 0, 0)),
            pl.BlockSpec((K, N), lambda i: (0, 0)),
            pl.BlockSpec((K, N), lambda i: (0, 0)),
            pl.BlockSpec((1, N), lambda i: (0, 0)),
        ],
        out_specs=pl.BlockSpec((_BT, H // 2, G, N), lambda i: (i, 0, 0, 0)),
        compiler_params=pltpu.CompilerParams(
            dimension_semantics=("parallel",),
            vmem_limit_bytes=_VMEM_LIMIT,
        ),
    )(cols, we, wo, bN)


def _mlp_body(x_ref, w1_ref, b1_ref, w2_ref, b2_ref, o_ref):
    h = jnp.dot(x_ref[...], w1_ref[...], preferred_element_type=jnp.float32)
    h = jnp.maximum(h + b1_ref[...], 0.0)
    o_ref[...] = jnp.dot(h, w2_ref[...], preferred_element_type=jnp.float32) + b2_ref[...]


def _mlp(x, w1, b1, w2, b2):
    B, K = x.shape
    N1 = w1.shape[1]
    N2 = w2.shape[1]
    return pl.pallas_call(
        _mlp_body,
        out_shape=jax.ShapeDtypeStruct((B, N2), jnp.float32),
        grid=(1,),
        in_specs=[
            pl.BlockSpec((B, K), lambda i: (0, 0)),
            pl.BlockSpec((K, N1), lambda i: (0, 0)),
            pl.BlockSpec((1, N1), lambda i: (0, 0)),
            pl.BlockSpec((N1, N2), lambda i: (0, 0)),
            pl.BlockSpec((1, N2), lambda i: (0, 0)),
        ],
        out_specs=pl.BlockSpec((B, N2), lambda i: (0, 0)),
        compiler_params=pltpu.CompilerParams(
            dimension_semantics=("arbitrary",),
            vmem_limit_bytes=_VMEM_LIMIT,
        ),
    )(x, w1, b1, w2, b2)


def kernel(x_nchw, wc1, bc1, wc2, bc2, wc3, bc3, wl1, bl1, wl2, bl2):
    B = x_nchw.shape[0]
    x = _conv1(x_nchw, wc1, bc1)
    x = x.reshape(B, 64, 64, 16)
    x = _packed_conv(x, wc2, bc2, p=16)
    x = x.reshape(B, 32, 32, 32)
    x = _packed_conv(x, wc3, bc3, p=8)
    x = x.reshape(B, -1)
    return _mlp(x, wl1.astype(_BF), bl1, wl2, bl2)


# speedup vs baseline: 5.4264x; 1.2517x over previous
import functools

import jax
import jax.numpy as jnp
from jax.experimental import pallas as pl
from jax.experimental.pallas import tpu as pltpu

_VMEM_LIMIT = 48 * 1024 * 1024
_BF = jnp.bfloat16


_BT = 16


def _conv1_body(x_ref, we_ref, wo_ref, b_ref, o_ref):
    dn = (((0,), (0,)), ((), ()))
    for bi in range(_BT):
        oe = None
        oo = None
        for dh in range(3):
            lhs_t = x_ref[bi, :, dh * 8:dh * 8 + 1024]
            pe = jax.lax.dot_general(lhs_t, we_ref[dh], dn,
                                     preferred_element_type=jnp.float32)
            po = jax.lax.dot_general(lhs_t, wo_ref[dh], dn,
                                     preferred_element_type=jnp.float32)
            oe = pe if oe is None else oe + pe
            oo = po if oo is None else oo + po
        z = jnp.maximum(jnp.maximum(oe, oo) + b_ref[...], 0.0)
        z = z.reshape(64, 2, 8, 128)
        z = jnp.maximum(z[:, 0], z[:, 1])
        o_ref[bi] = z.astype(o_ref.dtype)


def _conv1(x_nchw, w3, b):
    B = x_nchw.shape[0]
    xp = jnp.pad(x_nchw, ((0, 0), (0, 0), (1, 1), (1, 15))).astype(_BF)
    xg = xp.reshape(B, 3, 130, 9, 16)
    w18 = jnp.concatenate([xg[:, :, :, 0:8, :], xg[:, :, :, 1:9, 0:2]],
                          axis=-1)
    cols = w18.transpose(0, 4, 1, 2, 3).reshape(B, 54, 1040)

    w3b = w3.astype(_BF)
    we = jnp.stack([jnp.pad(w3b, ((0, 0), (2 * m, 15 - 2 * m), (0, 0), (0, 0)))
                    for m in range(8)], axis=3)
    wo = jnp.stack([jnp.pad(w3b, ((0, 0), (2 * m + 1, 14 - 2 * m), (0, 0), (0, 0)))
                    for m in range(8)], axis=3)
    we = we.reshape(3, 54, 128)
    wo = wo.reshape(3, 54, 128)
    b128 = jnp.tile(b, (1, 8))

    return pl.pallas_call(
        _conv1_body,
        out_shape=jax.ShapeDtypeStruct((B, 64, 8, 128), _BF),
        grid=(B // _BT,),
        in_specs=[
            pl.BlockSpec((_BT, 54, 1040), lambda i: (i, 0, 0)),
            pl.BlockSpec((3, 54, 128), lambda i: (0, 0, 0)),
            pl.BlockSpec((3, 54, 128), lambda i: (0, 0, 0)),
            pl.BlockSpec((1, 128), lambda i: (0, 0)),
        ],
        out_specs=pl.BlockSpec((_BT, 64, 8, 128), lambda i: (i, 0, 0, 0)),
        compiler_params=pltpu.CompilerParams(
            dimension_semantics=("parallel",),
            vmem_limit_bytes=_VMEM_LIMIT,
        ),
    )(cols, we, wo, b128)


def _packed_body(x_ref, we_ref, wo_ref, b_ref, o_ref, *, H, G, N):
    dn = (((0,), (0,)), ((), ()))
    for bi in range(_BT):
        oe = None
        oo = None
        for dh in range(3):
            lhs_t = x_ref[bi, :, dh * G:dh * G + H * G]
            pe = jax.lax.dot_general(lhs_t, we_ref[dh], dn,
                                     preferred_element_type=jnp.float32)
            po = jax.lax.dot_general(lhs_t, wo_ref[dh], dn,
                                     preferred_element_type=jnp.float32)
            oe = pe if oe is None else oe + pe
            oo = po if oo is None else oo + po
        z = jnp.maximum(jnp.maximum(oe, oo) + b_ref[...], 0.0)
        z = z.reshape(H // 2, 2, G, N)
        z = jnp.maximum(z[:, 0], z[:, 1])
        o_ref[bi] = z.astype(o_ref.dtype)


def _packed_conv(x, w3, b, *, p):
    B, H, W, C = x.shape
    Co = w3.shape[-1]
    G = W // p
    K = 3 * (p + 2) * C
    N = (p // 2) * Co
    xp = jnp.pad(x, ((0, 0), (1, 1), (1, (G + 1) * p - W - 1), (0, 0)))
    xg = xp.reshape(B, H + 2, G + 1, p, C)
    win = jnp.concatenate([xg[:, :, 0:G], xg[:, :, 1:G + 1, 0:2]],
                          axis=3)
    K2 = (p + 2) * C
    cols = win.transpose(0, 3, 4, 1, 2).reshape(B, K2, (H + 2) * G)

    w3b = w3.astype(_BF)
    we = jnp.stack([jnp.pad(w3b, ((0, 0), (2 * m, p - 1 - 2 * m), (0, 0), (0, 0)))
                    for m in range(p // 2)], axis=3)
    wo = jnp.stack([jnp.pad(w3b, ((0, 0), (2 * m + 1, p - 2 - 2 * m), (0, 0), (0, 0)))
                    for m in range(p // 2)], axis=3)
    we = we.reshape(3, K2, N)
    wo = wo.reshape(3, K2, N)
    bN = jnp.tile(b, (1, p // 2))

    body = functools.partial(_packed_body, H=H, G=G, N=N)
    return pl.pallas_call(
        body,
        out_shape=jax.ShapeDtypeStruct((B, H // 2, G, N), _BF),
        grid=(B // _BT,),
        in_specs=[
            pl.BlockSpec((_BT, K2, (H + 2) * G), lambda i: (i, 0, 0)),
            pl.BlockSpec((3, K2, N), lambda i: (0, 0, 0)),
            pl.BlockSpec((3, K2, N), lambda i: (0, 0, 0)),
            pl.BlockSpec((1, N), lambda i: (0, 0)),
        ],
        out_specs=pl.BlockSpec((_BT, H // 2, G, N), lambda i: (i, 0, 0, 0)),
        compiler_params=pltpu.CompilerParams(
            dimension_semantics=("parallel",),
            vmem_limit_bytes=_VMEM_LIMIT,
        ),
    )(cols, we, wo, bN)


def _mlp_body(x_ref, w1_ref, b1_ref, w2_ref, b2_ref, o_ref):
    h = jnp.dot(x_ref[...], w1_ref[...], preferred_element_type=jnp.float32)
    h = jnp.maximum(h + b1_ref[...], 0.0)
    o_ref[...] = jnp.dot(h, w2_ref[...], preferred_element_type=jnp.float32) + b2_ref[...]


def _mlp(x, w1, b1, w2, b2):
    B, K = x.shape
    N1 = w1.shape[1]
    N2 = w2.shape[1]
    return pl.pallas_call(
        _mlp_body,
        out_shape=jax.ShapeDtypeStruct((B, N2), jnp.float32),
        grid=(1,),
        in_specs=[
            pl.BlockSpec((B, K), lambda i: (0, 0)),
            pl.BlockSpec((K, N1), lambda i: (0, 0)),
            pl.BlockSpec((1, N1), lambda i: (0, 0)),
            pl.BlockSpec((N1, N2), lambda i: (0, 0)),
            pl.BlockSpec((1, N2), lambda i: (0, 0)),
        ],
        out_specs=pl.BlockSpec((B, N2), lambda i: (0, 0)),
        compiler_params=pltpu.CompilerParams(
            dimension_semantics=("arbitrary",),
            vmem_limit_bytes=_VMEM_LIMIT,
        ),
    )(x, w1, b1, w2, b2)


def kernel(x_nchw, wc1, bc1, wc2, bc2, wc3, bc3, wl1, bl1, wl2, bl2):
    B = x_nchw.shape[0]
    x = _conv1(x_nchw, wc1, bc1)
    x = x.reshape(B, 64, 64, 16)
    x = _packed_conv(x, wc2, bc2, p=16)
    x = x.reshape(B, 32, 32, 32)
    x = _packed_conv(x, wc3, bc3, p=8)
    x = x.reshape(B, -1)
    return _mlp(x, wl1.astype(_BF), bl1, wl2, bl2)
